# baseline (speedup 1.0000x reference)
import functools

import jax
import jax.numpy as jnp
from jax import lax
from jax.experimental import pallas as pl
from jax.experimental.pallas import tpu as pltpu
from jax.experimental.pallas import tpu_sc as plsc

B = 16384
VOCAB_CAT = 1000
CAT_EMB = 64
SUBCAT_EMB = 128
OUT_DIM = 128
NUM_SCALARS = 8

_NC = 2
_NS = 16
_NW = _NC * _NS
_BPW = B // _NW
_CHUNK = 128
_NCH = _BPW // _CHUNK
_IDX_ROWS = B // _CHUNK


@functools.lru_cache(maxsize=None)
def _build_sc_gather():
    mesh = plsc.VectorSubcoreMesh(core_axis_name="c", subcore_axis_name="s")

    @functools.partial(
        pl.kernel,
        mesh=mesh,
        out_type=[
            jax.ShapeDtypeStruct((B, SUBCAT_EMB), jnp.float32),
            jax.ShapeDtypeStruct((B, OUT_DIM), jnp.float32),
        ],
        scratch_types=[
            pltpu.VMEM((_NCH, _CHUNK), jnp.int32),
            pltpu.VMEM((_NCH, _CHUNK), jnp.int32),
            pltpu.VMEM((2 * _CHUNK, SUBCAT_EMB), jnp.float32),
            pltpu.VMEM((2 * _CHUNK, OUT_DIM), jnp.float32),
            pltpu.SemaphoreType.DMA,
            pltpu.SemaphoreType.DMA,
            pltpu.SemaphoreType.DMA,
            pltpu.SemaphoreType.DMA,
        ],
    )
    def gather_kernel(sub_idx_hbm, cat_idx_hbm, sub_tbl_hbm, cat_tbl_hbm,
                      se_hbm, ce_hbm, sidx, cidx, srows, crows,
                      gs_sem, gc_sem, ws_sem, wc_sem):
        wid = lax.axis_index("s") * _NC + lax.axis_index("c")
        idx_row0 = wid * _NCH
        pltpu.sync_copy(sub_idx_hbm.at[pl.ds(idx_row0, _NCH)], sidx)
        pltpu.sync_copy(cat_idx_hbm.at[pl.ds(idx_row0, _NCH)], cidx)
        base = wid * _BPW

        def gather(ch):
            slot = (ch % 2) * _CHUNK
            return (pltpu.async_copy(sub_tbl_hbm.at[sidx.at[ch]],
                                     srows.at[pl.ds(slot, _CHUNK)], gs_sem),
                    pltpu.async_copy(cat_tbl_hbm.at[cidx.at[ch]],
                                     crows.at[pl.ds(slot, _CHUNK)], gc_sem))

        def writeback(ch):
            slot = (ch % 2) * _CHUNK
            dst = base + ch * _CHUNK
            return (pltpu.async_copy(srows.at[pl.ds(slot, _CHUNK)],
                                     se_hbm.at[pl.ds(dst, _CHUNK)], ws_sem),
                    pltpu.async_copy(crows.at[pl.ds(slot, _CHUNK)],
                                     ce_hbm.at[pl.ds(dst, _CHUNK)], wc_sem))

        g = {0: gather(0)}
        w = {}
        for ch in range(_NCH):
            for c in g.pop(ch):
                c.wait()
            w[ch] = writeback(ch)
            if ch + 1 < _NCH:
                if ch >= 1:
                    for c in w.pop(ch - 1):
                        c.wait()
                g[ch + 1] = gather(ch + 1)
        for ch in sorted(w):
            for c in w.pop(ch):
                c.wait()

    return gather_kernel


def _fuse_cat_body(tblT_ref, w_ref, b_ref, o_ref):
    o_ref[...] = lax.dot_general(
        tblT_ref[...], w_ref[...], (((0,), (0,)), ((), ())),
        preferred_element_type=jnp.float32) + b_ref[...]


_BB = 2048


def _mlp_body(cef_ref, se_ref, sT_ref, ws_ref, wr_ref, o_ref):
    sT = sT_ref[...]
    row = lax.broadcasted_iota(jnp.int32, sT.shape, 0)
    sT = jnp.where(row == 0, jnp.log1p(sT), sT)
    h = cef_ref[...]
    h = h + jnp.dot(se_ref[...], ws_ref[...], preferred_element_type=jnp.float32)
    h = h + lax.dot_general(sT, wr_ref[...], (((0,), (0,)), ((), ())),
                            preferred_element_type=jnp.float32)
    o_ref[...] = 0.5 * h * (1.0 + lax.erf(h * 0.7071067811865476))


@jax.jit
def kernel(category_idx, subcategory_idx, total_weight, step_zscore,
           stage_coverage, mask_flags, cat_table, subcat_table, W, b):
    fused_cat = pl.pallas_call(
        _fuse_cat_body,
        in_specs=[
            pl.BlockSpec((CAT_EMB, VOCAB_CAT), lambda: (0, 0)),
            pl.BlockSpec((CAT_EMB, OUT_DIM), lambda: (0, 0)),
            pl.BlockSpec((1, OUT_DIM), lambda: (0, 0)),
        ],
        out_specs=pl.BlockSpec((VOCAB_CAT, OUT_DIM), lambda: (0, 0)),
        out_shape=jax.ShapeDtypeStruct((VOCAB_CAT, OUT_DIM), jnp.float32),
    )(cat_table.T, W[:CAT_EMB], b[None, :])

    sub_idx = subcategory_idx.astype(jnp.int32).reshape(_IDX_ROWS, _CHUNK)
    cat_idx = category_idx.astype(jnp.int32).reshape(_IDX_ROWS, _CHUNK)
    se, cef = _build_sc_gather()(sub_idx, cat_idx, subcat_table, fused_cat)

    sT = jnp.concatenate(
        [total_weight[None, :], step_zscore[None, :], stage_coverage[None, :],
         mask_flags.T], axis=0)
    ws = W[CAT_EMB:CAT_EMB + SUBCAT_EMB]
    wr = W[CAT_EMB + SUBCAT_EMB:]

    out = pl.pallas_call(
        _mlp_body,
        grid=(B // _BB,),
        in_specs=[
            pl.BlockSpec((_BB, OUT_DIM), lambda i: (i, 0)),
            pl.BlockSpec((_BB, SUBCAT_EMB), lambda i: (i, 0)),
            pl.BlockSpec((NUM_SCALARS, _BB), lambda i: (0, i)),
            pl.BlockSpec((SUBCAT_EMB, OUT_DIM), lambda i: (0, 0)),
            pl.BlockSpec((NUM_SCALARS, OUT_DIM), lambda i: (0, 0)),
        ],
        out_specs=pl.BlockSpec((_BB, OUT_DIM), lambda i: (i, 0)),
        out_shape=jax.ShapeDtypeStruct((B, OUT_DIM), jnp.float32),
    )(cef, se, sT, ws, wr)
    return out


# speedup vs baseline: 1.0389x; 1.0389x over previous
import functools

import jax
import jax.numpy as jnp
from jax import lax
from jax.experimental import pallas as pl
from jax.experimental.pallas import tpu as pltpu
from jax.experimental.pallas import tpu_sc as plsc

B = 16384
VOCAB_CAT = 1000
CAT_EMB = 64
SUBCAT_EMB = 128
OUT_DIM = 128
NUM_SCALARS = 8

_NC = 2
_NS = 16
_NW = _NC * _NS
_BPW = B // _NW
_CHUNK = 128
_NCH = _BPW // _CHUNK
_IDX_ROWS = B // _CHUNK


@functools.lru_cache(maxsize=None)
def _build_sc_gather():
    mesh = plsc.VectorSubcoreMesh(core_axis_name="c", subcore_axis_name="s")

    @functools.partial(
        pl.kernel,
        mesh=mesh,
        out_type=[
            jax.ShapeDtypeStruct((B, SUBCAT_EMB), jnp.float32),
            jax.ShapeDtypeStruct((B, OUT_DIM), jnp.float32),
        ],
        scratch_types=[
            pltpu.VMEM((_NCH, _CHUNK), jnp.int32),
            pltpu.VMEM((_NCH, _CHUNK), jnp.int32),
            pltpu.VMEM((2 * _CHUNK, SUBCAT_EMB), jnp.float32),
            pltpu.VMEM((2 * _CHUNK, OUT_DIM), jnp.float32),
            pltpu.SemaphoreType.DMA,
            pltpu.SemaphoreType.DMA,
            pltpu.SemaphoreType.DMA,
            pltpu.SemaphoreType.DMA,
        ],
    )
    def gather_kernel(sub_idx_hbm, cat_idx_hbm, sub_tbl_hbm, cat_tbl_hbm,
                      se_hbm, ce_hbm, sidx, cidx, srows, crows,
                      gs_sem, gc_sem, ws_sem, wc_sem):
        wid = lax.axis_index("s") * _NC + lax.axis_index("c")
        idx_row0 = wid * _NCH
        pltpu.sync_copy(sub_idx_hbm.at[pl.ds(idx_row0, _NCH)], sidx)
        pltpu.sync_copy(cat_idx_hbm.at[pl.ds(idx_row0, _NCH)], cidx)
        base = wid * _BPW

        def gather(ch):
            slot = (ch % 2) * _CHUNK
            return (pltpu.async_copy(sub_tbl_hbm.at[sidx.at[ch]],
                                     srows.at[pl.ds(slot, _CHUNK)], gs_sem),
                    pltpu.async_copy(cat_tbl_hbm.at[cidx.at[ch]],
                                     crows.at[pl.ds(slot, _CHUNK)], gc_sem))

        def writeback(ch):
            slot = (ch % 2) * _CHUNK
            dst = base + ch * _CHUNK
            return (pltpu.async_copy(srows.at[pl.ds(slot, _CHUNK)],
                                     se_hbm.at[pl.ds(dst, _CHUNK)], ws_sem),
                    pltpu.async_copy(crows.at[pl.ds(slot, _CHUNK)],
                                     ce_hbm.at[pl.ds(dst, _CHUNK)], wc_sem))

        g = {0: gather(0)}
        w = {}
        for ch in range(_NCH):
            for c in g.pop(ch):
                c.wait()
            w[ch] = writeback(ch)
            if ch + 1 < _NCH:
                if ch >= 1:
                    for c in w.pop(ch - 1):
                        c.wait()
                g[ch + 1] = gather(ch + 1)
        for ch in sorted(w):
            for c in w.pop(ch):
                c.wait()

    return gather_kernel


def _fuse_cat_body(tblT_ref, w_ref, b_ref, o_ref):
    o_ref[...] = lax.dot_general(
        tblT_ref[...], w_ref[...], (((0,), (0,)), ((), ())),
        preferred_element_type=jnp.float32) + b_ref[...]


_BB = 4096


def _mlp_body(cef_ref, se_ref, sT_ref, ws_ref, wr_ref, o_ref):
    sT = sT_ref[...]
    row = lax.broadcasted_iota(jnp.int32, sT.shape, 0)
    sT = jnp.where(row == 0, jnp.log1p(sT), sT)
    h = cef_ref[...]
    h = h + jnp.dot(se_ref[...], ws_ref[...], preferred_element_type=jnp.float32)
    h = h + lax.dot_general(sT, wr_ref[...], (((0,), (0,)), ((), ())),
                            preferred_element_type=jnp.float32)
    o_ref[...] = 0.5 * h * (1.0 + lax.erf(h * 0.7071067811865476))


@jax.jit
def kernel(category_idx, subcategory_idx, total_weight, step_zscore,
           stage_coverage, mask_flags, cat_table, subcat_table, W, b):
    fused_cat = pl.pallas_call(
        _fuse_cat_body,
        in_specs=[
            pl.BlockSpec((CAT_EMB, VOCAB_CAT), lambda: (0, 0)),
            pl.BlockSpec((CAT_EMB, OUT_DIM), lambda: (0, 0)),
            pl.BlockSpec((1, OUT_DIM), lambda: (0, 0)),
        ],
        out_specs=pl.BlockSpec((VOCAB_CAT, OUT_DIM), lambda: (0, 0)),
        out_shape=jax.ShapeDtypeStruct((VOCAB_CAT, OUT_DIM), jnp.float32),
    )(cat_table.T, W[:CAT_EMB], b[None, :])

    sub_idx = subcategory_idx.astype(jnp.int32).reshape(_IDX_ROWS, _CHUNK)
    cat_idx = category_idx.astype(jnp.int32).reshape(_IDX_ROWS, _CHUNK)
    se, cef = _build_sc_gather()(sub_idx, cat_idx, subcat_table, fused_cat)

    sT = jnp.concatenate(
        [total_weight[None, :], step_zscore[None, :], stage_coverage[None, :],
         mask_flags.T], axis=0)
    ws = W[CAT_EMB:CAT_EMB + SUBCAT_EMB]
    wr = W[CAT_EMB + SUBCAT_EMB:]

    out = pl.pallas_call(
        _mlp_body,
        grid=(B // _BB,),
        in_specs=[
            pl.BlockSpec((_BB, OUT_DIM), lambda i: (i, 0)),
            pl.BlockSpec((_BB, SUBCAT_EMB), lambda i: (i, 0)),
            pl.BlockSpec((NUM_SCALARS, _BB), lambda i: (0, i)),
            pl.BlockSpec((SUBCAT_EMB, OUT_DIM), lambda i: (0, 0)),
            pl.BlockSpec((NUM_SCALARS, OUT_DIM), lambda i: (0, 0)),
        ],
        out_specs=pl.BlockSpec((_BB, OUT_DIM), lambda i: (i, 0)),
        out_shape=jax.ShapeDtypeStruct((B, OUT_DIM), jnp.float32),
    )(cef, se, sT, ws, wr)
    return out


# speedup vs baseline: 1.0398x; 1.0008x over previous
import functools

import jax
import jax.numpy as jnp
from jax import lax
from jax.experimental import pallas as pl
from jax.experimental.pallas import tpu as pltpu
from jax.experimental.pallas import tpu_sc as plsc

B = 16384
VOCAB_CAT = 1000
CAT_EMB = 64
SUBCAT_EMB = 128
OUT_DIM = 128
NUM_SCALARS = 8

_NC = 2
_NS = 16
_NW = _NC * _NS
_BPW = B // _NW
_CHUNK = 128
_NCH = _BPW // _CHUNK
_IDX_ROWS = B // _CHUNK


@functools.lru_cache(maxsize=None)
def _build_sc_gather():
    mesh = plsc.VectorSubcoreMesh(core_axis_name="c", subcore_axis_name="s")

    @functools.partial(
        pl.kernel,
        mesh=mesh,
        out_type=[
            jax.ShapeDtypeStruct((B, SUBCAT_EMB), jnp.float32),
            jax.ShapeDtypeStruct((B, OUT_DIM), jnp.float32),
        ],
        scratch_types=[
            pltpu.VMEM((_NCH, _CHUNK), jnp.int32),
            pltpu.VMEM((_NCH, _CHUNK), jnp.int32),
            pltpu.VMEM((2 * _CHUNK, SUBCAT_EMB), jnp.float32),
            pltpu.VMEM((2 * _CHUNK, OUT_DIM), jnp.float32),
            pltpu.SemaphoreType.DMA,
            pltpu.SemaphoreType.DMA,
            pltpu.SemaphoreType.DMA,
            pltpu.SemaphoreType.DMA,
        ],
    )
    def gather_kernel(sub_idx_hbm, cat_idx_hbm, sub_tbl_hbm, cat_tbl_hbm,
                      se_hbm, ce_hbm, sidx, cidx, srows, crows,
                      gs_sem, gc_sem, ws_sem, wc_sem):
        wid = lax.axis_index("s") * _NC + lax.axis_index("c")
        idx_row0 = wid * _NCH
        pltpu.sync_copy(sub_idx_hbm.at[pl.ds(idx_row0, _NCH)], sidx)
        pltpu.sync_copy(cat_idx_hbm.at[pl.ds(idx_row0, _NCH)], cidx)
        base = wid * _BPW

        def gather(ch):
            slot = (ch % 2) * _CHUNK
            return (pltpu.async_copy(sub_tbl_hbm.at[sidx.at[ch]],
                                     srows.at[pl.ds(slot, _CHUNK)], gs_sem),
                    pltpu.async_copy(cat_tbl_hbm.at[cidx.at[ch]],
                                     crows.at[pl.ds(slot, _CHUNK)], gc_sem))

        def writeback(ch):
            slot = (ch % 2) * _CHUNK
            dst = base + ch * _CHUNK
            return (pltpu.async_copy(srows.at[pl.ds(slot, _CHUNK)],
                                     se_hbm.at[pl.ds(dst, _CHUNK)], ws_sem),
                    pltpu.async_copy(crows.at[pl.ds(slot, _CHUNK)],
                                     ce_hbm.at[pl.ds(dst, _CHUNK)], wc_sem))

        g = {0: gather(0)}
        w = {}
        for ch in range(_NCH):
            for c in g.pop(ch):
                c.wait()
            w[ch] = writeback(ch)
            if ch + 1 < _NCH:
                if ch >= 1:
                    for c in w.pop(ch - 1):
                        c.wait()
                g[ch + 1] = gather(ch + 1)
        for ch in sorted(w):
            for c in w.pop(ch):
                c.wait()

    return gather_kernel


def _fuse_cat_body(tblT_ref, w_ref, b_ref, o_ref):
    o_ref[...] = lax.dot_general(
        tblT_ref[...], w_ref[...], (((0,), (0,)), ((), ())),
        preferred_element_type=jnp.float32) + b_ref[...]


_BB = 4096


def _mlp_body(cef_ref, se_ref, sT_ref, wsa_ref, wsb_ref, wr_ref, o_ref):
    sT = sT_ref[...]
    row = lax.broadcasted_iota(jnp.int32, sT.shape, 0)
    sT = jnp.where(row == 0, jnp.log1p(sT), sT)
    se = se_ref[...]
    h = cef_ref[...]
    h = h + jnp.dot(se[:, :CAT_EMB], wsa_ref[...],
                    preferred_element_type=jnp.float32)
    h = h + jnp.dot(se[:, CAT_EMB:], wsb_ref[...],
                    preferred_element_type=jnp.float32)
    h = h + lax.dot_general(sT, wr_ref[...], (((0,), (0,)), ((), ())),
                            preferred_element_type=jnp.float32)
    o_ref[...] = 0.5 * h * (1.0 + lax.erf(h * 0.7071067811865476))


@jax.jit
def kernel(category_idx, subcategory_idx, total_weight, step_zscore,
           stage_coverage, mask_flags, cat_table, subcat_table, W, b):
    fused_cat = pl.pallas_call(
        _fuse_cat_body,
        grid=(1,),
        in_specs=[
            pl.BlockSpec((CAT_EMB, VOCAB_CAT), lambda i: (0, 0)),
            pl.BlockSpec((CAT_EMB, OUT_DIM), lambda i: (0, 0)),
            pl.BlockSpec((1, OUT_DIM), lambda i: (0, 0)),
        ],
        out_specs=pl.BlockSpec((VOCAB_CAT, OUT_DIM), lambda i: (0, 0)),
        out_shape=jax.ShapeDtypeStruct((VOCAB_CAT, OUT_DIM), jnp.float32),
    )(cat_table.T, W, b[None, :])

    sub_idx = subcategory_idx.astype(jnp.int32).reshape(_IDX_ROWS, _CHUNK)
    cat_idx = category_idx.astype(jnp.int32).reshape(_IDX_ROWS, _CHUNK)
    se, cef = _build_sc_gather()(sub_idx, cat_idx, subcat_table, fused_cat)

    sT = jnp.concatenate(
        [total_weight[None, :], step_zscore[None, :], stage_coverage[None, :],
         mask_flags.T], axis=0)

    out = pl.pallas_call(
        _mlp_body,
        grid=(B // _BB,),
        in_specs=[
            pl.BlockSpec((_BB, OUT_DIM), lambda i: (i, 0)),
            pl.BlockSpec((_BB, SUBCAT_EMB), lambda i: (i, 0)),
            pl.BlockSpec((NUM_SCALARS, _BB), lambda i: (0, i)),
            pl.BlockSpec((CAT_EMB, OUT_DIM), lambda i: (1, 0)),
            pl.BlockSpec((CAT_EMB, OUT_DIM), lambda i: (2, 0)),
            pl.BlockSpec((NUM_SCALARS, OUT_DIM), lambda i: (24, 0)),
        ],
        out_specs=pl.BlockSpec((_BB, OUT_DIM), lambda i: (i, 0)),
        out_shape=jax.ShapeDtypeStruct((B, OUT_DIM), jnp.float32),
    )(cef, se, sT, W, W, W)
    return out


# speedup vs baseline: 1.0426x; 1.0027x over previous
import functools

import jax
import jax.numpy as jnp
from jax import lax
from jax.experimental import pallas as pl
from jax.experimental.pallas import tpu as pltpu
from jax.experimental.pallas import tpu_sc as plsc

B = 16384
VOCAB_CAT = 1000
CAT_EMB = 64
SUBCAT_EMB = 128
OUT_DIM = 128
NUM_SCALARS = 8

_NC = 2
_NS = 16
_NW = _NC * _NS
_BPW = B // _NW
_CHUNK = 128
_NCH = _BPW // _CHUNK
_IDX_ROWS = B // _CHUNK


@functools.lru_cache(maxsize=None)
def _build_sc_gather():
    mesh = plsc.VectorSubcoreMesh(core_axis_name="c", subcore_axis_name="s")

    @functools.partial(
        pl.kernel,
        mesh=mesh,
        out_type=[
            jax.ShapeDtypeStruct((B, SUBCAT_EMB), jnp.float32),
            jax.ShapeDtypeStruct((B, OUT_DIM), jnp.float32),
        ],
        scratch_types=[
            pltpu.VMEM((_NCH, _CHUNK), jnp.int32),
            pltpu.VMEM((_NCH, _CHUNK), jnp.int32),
            pltpu.VMEM((2 * _CHUNK, SUBCAT_EMB), jnp.float32),
            pltpu.VMEM((2 * _CHUNK, OUT_DIM), jnp.float32),
            pltpu.SemaphoreType.DMA,
            pltpu.SemaphoreType.DMA,
            pltpu.SemaphoreType.DMA,
            pltpu.SemaphoreType.DMA,
        ],
    )
    def gather_kernel(sub_idx_hbm, cat_idx_hbm, sub_tbl_hbm, cat_tbl_hbm,
                      se_hbm, ce_hbm, sidx, cidx, srows, crows,
                      gs_sem, gc_sem, ws_sem, wc_sem):
        wid = lax.axis_index("s") * _NC + lax.axis_index("c")
        idx_row0 = wid * _NCH
        pltpu.sync_copy(sub_idx_hbm.at[pl.ds(idx_row0, _NCH)], sidx)
        pltpu.sync_copy(cat_idx_hbm.at[pl.ds(idx_row0, _NCH)], cidx)
        base = wid * _BPW

        def gather(ch):
            slot = (ch % 2) * _CHUNK
            return (pltpu.async_copy(sub_tbl_hbm.at[sidx.at[ch]],
                                     srows.at[pl.ds(slot, _CHUNK)], gs_sem),
                    pltpu.async_copy(cat_tbl_hbm.at[cidx.at[ch]],
                                     crows.at[pl.ds(slot, _CHUNK)], gc_sem))

        def writeback(ch):
            slot = (ch % 2) * _CHUNK
            dst = base + ch * _CHUNK
            return (pltpu.async_copy(srows.at[pl.ds(slot, _CHUNK)],
                                     se_hbm.at[pl.ds(dst, _CHUNK)], ws_sem),
                    pltpu.async_copy(crows.at[pl.ds(slot, _CHUNK)],
                                     ce_hbm.at[pl.ds(dst, _CHUNK)], wc_sem))

        g = {0: gather(0)}
        w = {}
        for ch in range(_NCH):
            for c in g.pop(ch):
                c.wait()
            w[ch] = writeback(ch)
            if ch + 1 < _NCH:
                if ch >= 1:
                    for c in w.pop(ch - 1):
                        c.wait()
                g[ch + 1] = gather(ch + 1)
        for ch in sorted(w):
            for c in w.pop(ch):
                c.wait()

    return gather_kernel


def _fuse_cat_body(tblT_ref, w_ref, b_ref, o_ref):
    o_ref[...] = lax.dot_general(
        tblT_ref[...], w_ref[...], (((0,), (0,)), ((), ())),
        preferred_element_type=jnp.float32) + b_ref[...]


_BB = 8192


def _mlp_body(cef_ref, se_ref, sT_ref, wsa_ref, wsb_ref, wr_ref, o_ref):
    sT = sT_ref[...]
    row = lax.broadcasted_iota(jnp.int32, sT.shape, 0)
    sT = jnp.where(row == 0, jnp.log1p(sT), sT)
    se = se_ref[...]
    h = cef_ref[...]
    h = h + jnp.dot(se[:, :CAT_EMB], wsa_ref[...],
                    preferred_element_type=jnp.float32)
    h = h + jnp.dot(se[:, CAT_EMB:], wsb_ref[...],
                    preferred_element_type=jnp.float32)
    h = h + lax.dot_general(sT, wr_ref[...], (((0,), (0,)), ((), ())),
                            preferred_element_type=jnp.float32)
    o_ref[...] = 0.5 * h * (1.0 + lax.erf(h * 0.7071067811865476))


@jax.jit
def kernel(category_idx, subcategory_idx, total_weight, step_zscore,
           stage_coverage, mask_flags, cat_table, subcat_table, W, b):
    fused_cat = pl.pallas_call(
        _fuse_cat_body,
        grid=(1,),
        in_specs=[
            pl.BlockSpec((CAT_EMB, VOCAB_CAT), lambda i: (0, 0)),
            pl.BlockSpec((CAT_EMB, OUT_DIM), lambda i: (0, 0)),
            pl.BlockSpec((1, OUT_DIM), lambda i: (0, 0)),
        ],
        out_specs=pl.BlockSpec((VOCAB_CAT, OUT_DIM), lambda i: (0, 0)),
        out_shape=jax.ShapeDtypeStruct((VOCAB_CAT, OUT_DIM), jnp.float32),
    )(cat_table.T, W, b[None, :])

    sub_idx = subcategory_idx.astype(jnp.int32).reshape(_IDX_ROWS, _CHUNK)
    cat_idx = category_idx.astype(jnp.int32).reshape(_IDX_ROWS, _CHUNK)
    se, cef = _build_sc_gather()(sub_idx, cat_idx, subcat_table, fused_cat)

    sT = jnp.concatenate(
        [total_weight[None, :], step_zscore[None, :], stage_coverage[None, :],
         mask_flags.T], axis=0)

    out = pl.pallas_call(
        _mlp_body,
        grid=(B // _BB,),
        in_specs=[
            pl.BlockSpec((_BB, OUT_DIM), lambda i: (i, 0)),
            pl.BlockSpec((_BB, SUBCAT_EMB), lambda i: (i, 0)),
            pl.BlockSpec((NUM_SCALARS, _BB), lambda i: (0, i)),
            pl.BlockSpec((CAT_EMB, OUT_DIM), lambda i: (1, 0)),
            pl.BlockSpec((CAT_EMB, OUT_DIM), lambda i: (2, 0)),
            pl.BlockSpec((NUM_SCALARS, OUT_DIM), lambda i: (24, 0)),
        ],
        out_specs=pl.BlockSpec((_BB, OUT_DIM), lambda i: (i, 0)),
        out_shape=jax.ShapeDtypeStruct((B, OUT_DIM), jnp.float32),
    )(cef, se, sT, W, W, W)
    return out
